# Initial kernel scaffold; baseline (speedup 1.0000x reference)
import jax
import jax.numpy as jnp
from jax.experimental import pallas as pl


def kernel(atomic_numbers, zs):
    raise NotImplementedError("write your pallas kernel here")


# speedup vs baseline: 5.5277x; 5.5277x over previous
import functools

import jax
import jax.numpy as jnp
from jax import lax
from jax.experimental import pallas as pl
from jax.experimental.pallas import tpu as pltpu
from jax.experimental.pallas import tpu_sc as plsc

_NC, _NS = 2, 16
_NW = _NC * _NS
_CHUNK = 128
_K = 8
_TW = 16
_TZ = 64
_BIG = 1 << 20


def _encode_table_body(zs_ref, t_ref, *, C):
    zs_b = jnp.broadcast_to(zs_ref[0:1, :], (_TZ, _TW))
    zrow = lax.broadcasted_iota(jnp.int32, (_TZ, _TW), 0)
    lane = lax.broadcasted_iota(jnp.int32, (_TZ, _TW), 1)
    j = jnp.sum((zs_b < zrow).astype(jnp.int32), axis=1, keepdims=True)
    j = jnp.minimum(j, C - 1)
    exact = jnp.sum((zs_b == zrow).astype(jnp.int32), axis=1, keepdims=True) > 0
    lo = jnp.maximum(j - 1, 0)
    zs_f = zs_b.astype(jnp.float32)
    zf = lax.broadcasted_iota(jnp.int32, (_TZ, 1), 0).astype(jnp.float32)
    zs_hi = jnp.sum(jnp.where(lane == j, zs_f, 0.0), axis=1, keepdims=True)
    zs_lo = jnp.sum(jnp.where(lane == lo, zs_f, 0.0), axis=1, keepdims=True)
    denom = jnp.maximum(zs_hi - zs_lo, 1.0)
    w_lo = (zs_hi - zf) / denom
    w_hi = (zf - zs_lo) / denom
    onehot = (lane == j).astype(jnp.float32)
    interp = jnp.where(lane == lo, w_lo, 0.0) + jnp.where(lane == j, w_hi, 0.0)
    t_ref[...] = jnp.where(exact, onehot, interp)


def _encode_table(zs_tc, C):
    return pl.pallas_call(
        functools.partial(_encode_table_body, C=C),
        out_shape=jax.ShapeDtypeStruct((_TZ, _TW), jnp.float32),
    )(zs_tc)


@functools.lru_cache(maxsize=None)
def _sc_gather(cpt):
    npad = cpt * _NW * _CHUNK
    zpt = cpt * _CHUNK
    mesh = plsc.VectorSubcoreMesh(core_axis_name="c", subcore_axis_name="s")

    @functools.partial(
        pl.kernel,
        out_type=jax.ShapeDtypeStruct((npad, _TW), jnp.float32),
        mesh=mesh,
        compiler_params=pltpu.CompilerParams(use_tc_tiling_on_sc=False),
        scratch_types=(
            [pltpu.VMEM((zpt,), jnp.int32)]
            + [pltpu.VMEM((_CHUNK, _TW), jnp.float32) for _ in range(_K)]
            + [pltpu.SemaphoreType.DMA, pltpu.SemaphoreType.DMA]
        ),
    )
    def gather_k(table_hbm, z_hbm, out_hbm, z_v, *rest):
        rows = rest[:_K]
        gsem, wsem = rest[_K], rest[_K + 1]
        wid = lax.axis_index("s") * _NC + lax.axis_index("c")
        pltpu.sync_copy(z_hbm.at[pl.ds(wid * zpt, zpt)], z_v)

        def superstep(t, carry):
            i0 = t * _K
            gs = [
                pltpu.async_copy(
                    table_hbm.at[z_v.at[pl.ds((i0 + b) * _CHUNK, _CHUNK)]],
                    rows[b],
                    gsem,
                )
                for b in range(_K)
            ]
            for g in gs:
                g.wait()
            ws = [
                pltpu.async_copy(
                    rows[b],
                    out_hbm.at[pl.ds((wid * cpt + i0 + b) * _CHUNK, _CHUNK)],
                    wsem,
                )
                for b in range(_K)
            ]
            for w in ws:
                w.wait()
            return carry

        lax.fori_loop(0, cpt // _K, superstep, 0)

    return gather_k


def kernel(atomic_numbers, zs):
    n = atomic_numbers.shape[0]
    C = zs.shape[0]
    nchunks = -(-n // _CHUNK)
    cpt = -(-nchunks // _NW)
    cpt = -(-cpt // _K) * _K
    npad = cpt * _NW * _CHUNK
    z_pad = jnp.pad(atomic_numbers.astype(jnp.int32), (0, npad - n))
    zs_pad = jnp.pad(zs.astype(jnp.int32), (0, _TW - C), constant_values=_BIG)
    zs_tc = jnp.broadcast_to(zs_pad[None, :], (8, _TW))
    table = _encode_table(zs_tc, C)
    out16 = _sc_gather(cpt)(table, z_pad)
    return out16[:n, :C]
